# Initial kernel scaffold; baseline (speedup 1.0000x reference)
import jax
import jax.numpy as jnp
from jax.experimental import pallas as pl


def kernel(audio_feat, params, token_emb):
    raise NotImplementedError("write your pallas kernel here")


# speedup vs baseline: 4.1846x; 4.1846x over previous
import functools

import jax
import jax.numpy as jnp
from jax import lax
from jax.experimental import pallas as pl
from jax.experimental.pallas import tpu as pltpu

_B, _T, _DA = 16, 512, 768
_KW, _DT, _VOCAB = 8, 512, 8192
_H, _DH, _FF = 12, 64, 3072
_EPS = 1e-5
_SQ = 16
_NQ = 1 + _KW

_bf16 = jnp.bfloat16
_f32 = jnp.float32


def _ln(x, g, b):
    m = jnp.mean(x, axis=-1, keepdims=True)
    v = jnp.mean((x - m) ** 2, axis=-1, keepdims=True)
    return (x - m) / jnp.sqrt(v + _EPS) * g + b


def _attn_kernel(a_ref, cls_ref, wq_ref, bq_ref, wkv_ref, bkv_ref,
                 wo_ref, bo_ref, g1_ref, be1_ref, w1_ref, b1_ref,
                 w2_ref, b2_ref, g2_ref, be2_ref,
                 pjw_ref, pjb_ref, ppw_ref, ppb_ref,
                 p_out_ref, kw_out_ref):
    cn = (((1,), (1,)), ((), ()))
    cm = (((1,), (0,)), ((), ()))

    xa = a_ref[0].astype(_bf16)
    wkv = wkv_ref[...]
    kva = lax.dot_general(xa, wkv, cm, preferred_element_type=_f32)
    kva = kva + bkv_ref[...]
    cls = cls_ref[...]
    kvc = lax.dot_general(cls.astype(_bf16), wkv, cm,
                          preferred_element_type=_f32) + bkv_ref[...]

    k_a = kva[:, :_DA].astype(_bf16)
    v_a = kva[:, _DA:].astype(_bf16)
    k_c = kvc[:_NQ, :_DA].astype(_bf16)
    v_c = kvc[:_NQ, _DA:].astype(_bf16)

    q = (jnp.dot(cls, wq_ref[...], preferred_element_type=_f32)
         + bq_ref[...]) * (1.0 / 8.0)

    hm = (lax.broadcasted_iota(jnp.int32, (_H, 1, _DA), 2) // _DH
          == lax.broadcasted_iota(jnp.int32, (_H, 1, _DA), 0))
    q3 = jnp.broadcast_to(q[None], (_H, _SQ, _DA))
    qz = jnp.where(hm, q3, 0.0).reshape(_H * _SQ, _DA).astype(_bf16)

    s1 = lax.dot_general(qz, k_c, cn, preferred_element_type=_f32)
    s2 = lax.dot_general(qz, k_a, cn, preferred_element_type=_f32)
    m = jnp.maximum(jnp.max(s1, -1, keepdims=True),
                    jnp.max(s2, -1, keepdims=True))
    e1 = jnp.exp(s1 - m)
    e2 = jnp.exp(s2 - m)
    denom = jnp.sum(e1, -1, keepdims=True) + jnp.sum(e2, -1, keepdims=True)
    o = (lax.dot_general(e1.astype(_bf16), v_c, cm, preferred_element_type=_f32)
         + lax.dot_general(e2.astype(_bf16), v_a, cm,
                           preferred_element_type=_f32)) / denom
    o16 = jnp.sum(jnp.where(hm, o.reshape(_H, _SQ, _DA), 0.0), axis=0)

    x1 = cls + jnp.dot(o16, wo_ref[...], preferred_element_type=_f32) + bo_ref[...]
    xn = _ln(x1, g1_ref[...], be1_ref[...])
    h = jax.nn.gelu(jnp.dot(xn, w1_ref[...], preferred_element_type=_f32)
                    + b1_ref[...])
    x2 = xn + jnp.dot(h, w2_ref[...], preferred_element_type=_f32) + b2_ref[...]
    xo = _ln(x2, g2_ref[...], be2_ref[...])

    yp = jnp.dot(xo[0:1], ppw_ref[...], preferred_element_type=_f32) + ppb_ref[...]
    ykw = jnp.dot(xo[1:1 + _KW], pjw_ref[...], preferred_element_type=_f32) + pjb_ref[...]
    p_out_ref[...] = yp.reshape(1, 1, _DT)
    kw_out_ref[...] = ykw.reshape(1, _KW, _DT)


def _vq_kernel(kw_ref, g_ref, b_ref, te_ref, out_ref):
    kw = kw_ref[...]
    mu = jnp.mean(kw, axis=0, keepdims=True)
    var = jnp.mean((kw - mu) ** 2, axis=0, keepdims=True)
    kwn = (kw - mu) / jnp.sqrt(var + _EPS) * g_ref[...] + b_ref[...]
    kn = kwn / (jnp.sqrt(jnp.sum(kwn * kwn, -1, keepdims=True)) + 1e-8)
    kn2 = kn.reshape(_B * _KW, _DT).astype(_bf16)

    te = te_ref[...]
    te32 = te.astype(_f32)
    tinv = 1.0 / (jnp.sqrt(jnp.sum(te32 * te32, -1, keepdims=True)) + 1e-8)
    tn = (te32 * tinv).astype(_bf16)

    cos = lax.dot_general(kn2, tn, (((1,), (1,)), ((), ())),
                          preferred_element_type=_f32)
    prob = jax.nn.softmax(cos, axis=-1)
    kws = lax.dot_general(prob.astype(_bf16), te, (((1,), (0,)), ((), ())),
                          preferred_element_type=_f32)
    out_ref[...] = kws.reshape(_B, _KW, _DT)


def _const(shape):
    nd = len(shape)
    return pl.BlockSpec(shape, lambda b: (0,) * nd)


@functools.partial(jax.jit)
def kernel(audio_feat, params, token_emb):
    p = params
    cls9 = jnp.concatenate([p['parallel_cls'][0], p['cascaded_cls'][0]], axis=0)
    cls16 = jnp.pad(cls9, ((0, _SQ - _NQ), (0, 0)))
    wkv = jnp.concatenate([p['Wk'], p['Wv']], axis=1).astype(_bf16)
    bkv = jnp.concatenate([p['bk'], p['bv']]).reshape(1, 2 * _DA)

    row = lambda a: a.reshape(1, -1)

    p_out, kw_raw = pl.pallas_call(
        _attn_kernel,
        grid=(_B,),
        in_specs=[
            pl.BlockSpec((1, _T, _DA), lambda b: (b, 0, 0)),
            _const((_SQ, _DA)),
            _const((_DA, _DA)), _const((1, _DA)),
            _const((_DA, 2 * _DA)), _const((1, 2 * _DA)),
            _const((_DA, _DA)), _const((1, _DA)),
            _const((1, _DA)), _const((1, _DA)),
            _const((_DA, _FF)), _const((1, _FF)),
            _const((_FF, _DA)), _const((1, _DA)),
            _const((1, _DA)), _const((1, _DA)),
            _const((_DA, _DT)), _const((1, _DT)),
            _const((_DA, _DT)), _const((1, _DT)),
        ],
        out_specs=[
            pl.BlockSpec((1, 1, _DT), lambda b: (b, 0, 0)),
            pl.BlockSpec((1, _KW, _DT), lambda b: (b, 0, 0)),
        ],
        out_shape=[
            jax.ShapeDtypeStruct((_B, 1, _DT), _f32),
            jax.ShapeDtypeStruct((_B, _KW, _DT), _f32),
        ],
        compiler_params=pltpu.CompilerParams(
            dimension_semantics=("arbitrary",)),
    )(audio_feat, cls16,
      p['Wq'], row(p['bq']), wkv, bkv,
      p['Wo'], row(p['bo']),
      row(p['ln1_g']), row(p['ln1_b']),
      p['ffn_W1'], row(p['ffn_b1']), p['ffn_W2'], row(p['ffn_b2']),
      row(p['ln2_g']), row(p['ln2_b']),
      p['proj_W'], row(p['proj_b']), p['pproj_W'], row(p['pproj_b']))

    keywords = pl.pallas_call(
        _vq_kernel,
        in_specs=[
            pl.BlockSpec((_B, _KW, _DT), lambda: (0, 0, 0)),
            pl.BlockSpec((1, 1, _DT), lambda: (0, 0, 0)),
            pl.BlockSpec((1, 1, _DT), lambda: (0, 0, 0)),
            pl.BlockSpec((_VOCAB, _DT), lambda: (0, 0)),
        ],
        out_specs=pl.BlockSpec((_B, _KW, _DT), lambda: (0, 0, 0)),
        out_shape=jax.ShapeDtypeStruct((_B, _KW, _DT), _f32),
    )(kw_raw, p['bn_g'].reshape(1, 1, _DT), p['bn_b'].reshape(1, 1, _DT),
      token_emb.astype(_bf16))

    return jnp.concatenate([p_out, keywords], axis=1)


# speedup vs baseline: 7.7597x; 1.8543x over previous
import functools

import jax
import jax.numpy as jnp
from jax import lax
from jax.experimental import pallas as pl
from jax.experimental.pallas import tpu as pltpu

_B, _T, _DA = 16, 512, 768
_KW, _DT, _VOCAB = 8, 512, 8192
_H, _DH, _FF = 12, 64, 3072
_EPS = 1e-5
_SQ = 16
_NQ = 1 + _KW
_R = _H * _SQ

_bf16 = jnp.bfloat16
_f32 = jnp.float32

_CN = (((1,), (1,)), ((), ()))
_CM = (((1,), (0,)), ((), ()))


def _ln(x, g, b):
    m = jnp.mean(x, axis=-1, keepdims=True)
    v = jnp.mean((x - m) ** 2, axis=-1, keepdims=True)
    return (x - m) / jnp.sqrt(v + _EPS) * g + b


def _head_mask(shape, row_axis, col_axis):
    return (lax.broadcasted_iota(jnp.int32, shape, col_axis) // _DH
            == lax.broadcasted_iota(jnp.int32, shape, row_axis))


def _ctx_kernel(a_ref, cls_ref, wq_ref, bq_ref, wk_ref, c_out_ref, qzk_s):
    @pl.when(pl.program_id(0) == 0)
    def _init():
        cls = cls_ref[...]
        q = (jnp.dot(cls, wq_ref[...], preferred_element_type=_f32)
             + bq_ref[...]) * (1.0 / 8.0)
        hm = _head_mask((_H, 1, _DA), 0, 2)
        qz = jnp.where(hm, jnp.broadcast_to(q[None], (_H, _SQ, _DA)), 0.0)
        qz = qz.reshape(_R, _DA).astype(_bf16)
        qzk = lax.dot_general(qz, wk_ref[...].astype(_bf16), _CN,
                              preferred_element_type=_f32)
        qzk_s[...] = qzk.astype(_bf16)

    qzk = qzk_s[...]
    xa = a_ref[0].astype(_bf16)
    clsx = cls_ref[: _NQ].astype(_bf16)
    s1 = lax.dot_general(qzk, clsx, _CN, preferred_element_type=_f32)
    s2 = lax.dot_general(qzk, xa, _CN, preferred_element_type=_f32)
    m = jnp.maximum(jnp.max(s1, -1, keepdims=True),
                    jnp.max(s2, -1, keepdims=True))
    e1 = jnp.exp(s1 - m)
    e2 = jnp.exp(s2 - m)
    den = jnp.sum(e1, -1, keepdims=True) + jnp.sum(e2, -1, keepdims=True)
    c = (lax.dot_general(e1.astype(_bf16), clsx, _CM, preferred_element_type=_f32)
         + lax.dot_general(e2.astype(_bf16), xa, _CM,
                           preferred_element_type=_f32)) / den
    c_out_ref[...] = c.astype(_bf16).reshape(1, _R, _DA)


def _tail_kernel(call_ref, cls_ref, wv_ref, bv_ref, wo_ref, bo_ref,
                 g1_ref, be1_ref, w1_ref, b1_ref, w2_ref, b2_ref,
                 g2_ref, be2_ref, pjw_ref, pjb_ref, ppw_ref, ppb_ref,
                 bng_ref, bnb_ref, te_ref, out_ref):
    c2 = call_ref[...].reshape(_B * _R, _DA)
    cw = lax.dot_general(c2, wv_ref[...].astype(_bf16), _CM,
                         preferred_element_type=_f32)
    hm4 = _head_mask((1, _H, 1, _DA), 1, 3)
    o = jnp.sum(jnp.where(hm4, cw.reshape(_B, _H, _SQ, _DA), 0.0),
                axis=1)
    o2 = o.reshape(_B * _SQ, _DA) + bv_ref[...]
    cls256 = jnp.broadcast_to(cls_ref[None], (_B, _SQ, _DA)).reshape(
        _B * _SQ, _DA)
    x1 = cls256 + jnp.dot(o2.astype(_bf16), wo_ref[...].astype(_bf16),
                          preferred_element_type=_f32) + bo_ref[...]
    xn = _ln(x1, g1_ref[...], be1_ref[...])
    h = jax.nn.gelu(jnp.dot(xn.astype(_bf16), w1_ref[...].astype(_bf16),
                            preferred_element_type=_f32) + b1_ref[...])
    x2 = xn + jnp.dot(h.astype(_bf16), w2_ref[...].astype(_bf16),
                      preferred_element_type=_f32) + b2_ref[...]
    xo = _ln(x2, g2_ref[...], be2_ref[...])
    xob = xo.astype(_bf16)

    yp = jnp.dot(xob, ppw_ref[...].astype(_bf16),
                 preferred_element_type=_f32) + ppb_ref[...]
    ykw = jnp.dot(xob, pjw_ref[...].astype(_bf16),
                  preferred_element_type=_f32) + pjb_ref[...]
    out_ref[:, 0:1, :] = yp.reshape(_B, _SQ, _DT)[:, 0:1, :]
    kw = ykw.reshape(_B, _SQ, _DT)[:, 1:_NQ, :]

    mu = jnp.mean(kw, axis=0, keepdims=True)
    var = jnp.mean((kw - mu) ** 2, axis=0, keepdims=True)
    kwn = (kw - mu) / jnp.sqrt(var + _EPS) * bng_ref[...] + bnb_ref[...]
    kn = kwn / (jnp.sqrt(jnp.sum(kwn * kwn, -1, keepdims=True)) + 1e-8)
    kn2 = kn.reshape(_B * _KW, _DT).astype(_bf16)

    te32 = te_ref[...]
    tinv = 1.0 / (jnp.sqrt(jnp.sum(te32 * te32, -1, keepdims=True)) + 1e-8)
    tn = (te32 * tinv).astype(_bf16)
    cos = lax.dot_general(kn2, tn, _CN, preferred_element_type=_f32)
    prob = jax.nn.softmax(cos, axis=-1)
    kws = lax.dot_general(prob.astype(_bf16), te32.astype(_bf16), _CM,
                          preferred_element_type=_f32)
    out_ref[:, 1:_NQ, :] = kws.reshape(_B, _KW, _DT)


def _const(shape):
    nd = len(shape)
    return pl.BlockSpec(shape, lambda b: (0,) * nd)


def _whole(shape):
    nd = len(shape)
    return pl.BlockSpec(shape, lambda: (0,) * nd)


@functools.partial(jax.jit)
def kernel(audio_feat, params, token_emb):
    p = params
    cls9 = jnp.concatenate([p['parallel_cls'][0], p['cascaded_cls'][0]], axis=0)
    cls16 = jnp.pad(cls9, ((0, _SQ - _NQ), (0, 0)))
    row = lambda a: a.reshape(1, -1)

    c_all = pl.pallas_call(
        _ctx_kernel,
        grid=(_B,),
        in_specs=[
            pl.BlockSpec((1, _T, _DA), lambda b: (b, 0, 0)),
            _const((_SQ, _DA)),
            _const((_DA, _DA)), _const((1, _DA)),
            _const((_DA, _DA)),
        ],
        out_specs=pl.BlockSpec((1, _R, _DA), lambda b: (b, 0, 0)),
        out_shape=jax.ShapeDtypeStruct((_B, _R, _DA), _bf16),
        scratch_shapes=[pltpu.VMEM((_R, _DA), _bf16)],
        compiler_params=pltpu.CompilerParams(
            dimension_semantics=("arbitrary",)),
    )(audio_feat, cls16, p['Wq'], row(p['bq']), p['Wk'])

    out = pl.pallas_call(
        _tail_kernel,
        in_specs=[
            _whole((_B, _R, _DA)),
            _whole((_SQ, _DA)),
            _whole((_DA, _DA)), _whole((1, _DA)),
            _whole((_DA, _DA)), _whole((1, _DA)),
            _whole((1, _DA)), _whole((1, _DA)),
            _whole((_DA, _FF)), _whole((1, _FF)),
            _whole((_FF, _DA)), _whole((1, _DA)),
            _whole((1, _DA)), _whole((1, _DA)),
            _whole((_DA, _DT)), _whole((1, _DT)),
            _whole((_DA, _DT)), _whole((1, _DT)),
            _whole((1, 1, _DT)), _whole((1, 1, _DT)),
            _whole((_VOCAB, _DT)),
        ],
        out_specs=_whole((_B, _NQ, _DT)),
        out_shape=jax.ShapeDtypeStruct((_B, _NQ, _DT), _f32),
    )(c_all, cls16,
      p['Wv'], row(p['bv']), p['Wo'], row(p['bo']),
      row(p['ln1_g']), row(p['ln1_b']),
      p['ffn_W1'], row(p['ffn_b1']), p['ffn_W2'], row(p['ffn_b2']),
      row(p['ln2_g']), row(p['ln2_b']),
      p['proj_W'], row(p['proj_b']), p['pproj_W'], row(p['pproj_b']),
      p['bn_g'].reshape(1, 1, _DT), p['bn_b'].reshape(1, 1, _DT),
      token_emb)

    return out


# speedup vs baseline: 8.1481x; 1.0500x over previous
import functools

import jax
import jax.numpy as jnp
from jax import lax
from jax.experimental import pallas as pl
from jax.experimental.pallas import tpu as pltpu

_B, _T, _DA = 16, 512, 768
_KW, _DT, _VOCAB = 8, 512, 8192
_H, _DH, _FF = 12, 64, 3072
_EPS = 1e-5
_SQ = 16
_NQ = 1 + _KW
_R = _H * _SQ

_bf16 = jnp.bfloat16
_f32 = jnp.float32

_CN = (((1,), (1,)), ((), ()))
_CM = (((1,), (0,)), ((), ()))


def _ln(x, g, b):
    m = jnp.mean(x, axis=-1, keepdims=True)
    v = jnp.mean((x - m) ** 2, axis=-1, keepdims=True)
    return (x - m) / jnp.sqrt(v + _EPS) * g + b


def _head_mask(shape, row_axis, col_axis):
    return (lax.broadcasted_iota(jnp.int32, shape, col_axis) // _DH
            == lax.broadcasted_iota(jnp.int32, shape, row_axis))


_BB = 2


def _ctx_kernel(a_ref, cls_ref, wq_ref, bq_ref, wk_ref, c_out_ref,
                qzk_s, s1_s):
    @pl.when(pl.program_id(0) == 0)
    def _init():
        cls = cls_ref[...]
        q = (jnp.dot(cls, wq_ref[...], preferred_element_type=_f32)
             + bq_ref[...]) * (1.0 / 8.0)
        hm = _head_mask((_H, 1, _DA), 0, 2)
        qz = jnp.where(hm, jnp.broadcast_to(q[None], (_H, _SQ, _DA)), 0.0)
        qz = qz.reshape(_R, _DA).astype(_bf16)
        qzk = lax.dot_general(qz, wk_ref[...].astype(_bf16), _CN,
                              preferred_element_type=_f32)
        qzk_s[...] = qzk.astype(_bf16)
        s1_s[...] = lax.dot_general(qzk_s[...], cls.astype(_bf16), _CN,
                                    preferred_element_type=_f32)

    qzk = qzk_s[...]
    s1 = s1_s[...][:, : _NQ]
    m1 = jnp.max(s1, -1, keepdims=True)
    clsx = cls_ref[: _NQ].astype(_bf16)
    for i in range(_BB):
        xa = a_ref[i].astype(_bf16)
        s2 = lax.dot_general(qzk, xa, _CN, preferred_element_type=_f32)
        m = jnp.maximum(m1, jnp.max(s2, -1, keepdims=True))
        e1 = jnp.exp(s1 - m)
        e2 = jnp.exp(s2 - m)
        den = jnp.sum(e1, -1, keepdims=True) + jnp.sum(e2, -1, keepdims=True)
        c = (lax.dot_general(e1.astype(_bf16), clsx, _CM,
                             preferred_element_type=_f32)
             + lax.dot_general(e2.astype(_bf16), xa, _CM,
                               preferred_element_type=_f32)) / den
        c_out_ref[i] = c.astype(_bf16)


def _tail_kernel(call_ref, cls_ref, wv_ref, bv_ref, wo_ref, bo_ref,
                 g1_ref, be1_ref, w1_ref, b1_ref, w2_ref, b2_ref,
                 g2_ref, be2_ref, pjw_ref, pjb_ref, ppw_ref, ppb_ref,
                 p_out_ref, kwr_ref):
    c2 = call_ref[...].reshape(_B * _R, _DA)
    cw = lax.dot_general(c2, wv_ref[...].astype(_bf16), _CM,
                         preferred_element_type=_f32)
    hm4 = _head_mask((1, _H, 1, _DA), 1, 3)
    o = jnp.sum(jnp.where(hm4, cw.reshape(_B, _H, _SQ, _DA), 0.0),
                axis=1)
    o2 = o.reshape(_B * _SQ, _DA) + bv_ref[...]
    cls256 = jnp.broadcast_to(cls_ref[None], (_B, _SQ, _DA)).reshape(
        _B * _SQ, _DA)
    x1 = cls256 + jnp.dot(o2.astype(_bf16), wo_ref[...].astype(_bf16),
                          preferred_element_type=_f32) + bo_ref[...]
    xn = _ln(x1, g1_ref[...], be1_ref[...])
    h = jax.nn.gelu(jnp.dot(xn.astype(_bf16), w1_ref[...].astype(_bf16),
                            preferred_element_type=_f32) + b1_ref[...])
    x2 = xn + jnp.dot(h.astype(_bf16), w2_ref[...].astype(_bf16),
                      preferred_element_type=_f32) + b2_ref[...]
    xo = _ln(x2, g2_ref[...], be2_ref[...])
    xob = xo.astype(_bf16)

    yp = jnp.dot(xob, ppw_ref[...].astype(_bf16),
                 preferred_element_type=_f32) + ppb_ref[...]
    ykw = jnp.dot(xob, pjw_ref[...].astype(_bf16),
                  preferred_element_type=_f32) + pjb_ref[...]
    p_out_ref[...] = yp.reshape(_B, _SQ, _DT)[:, 0:1, :]
    kwr_ref[...] = ykw.reshape(_B, _SQ, _DT)[:, 1:_NQ, :]


_VC = 1024
_NVC = _VOCAB // _VC


def _vq_kernel(kwr_ref, bng_ref, bnb_ref, te_ref, out_ref,
               kn_s, m_s, den_s, acc_s):
    v = pl.program_id(0)

    @pl.when(v == 0)
    def _init():
        kw = kwr_ref[...]
        mu = jnp.mean(kw, axis=0, keepdims=True)
        var = jnp.mean((kw - mu) ** 2, axis=0, keepdims=True)
        kwn = (kw - mu) / jnp.sqrt(var + _EPS) * bng_ref[...] + bnb_ref[...]
        kn = kwn / (jnp.sqrt(jnp.sum(kwn * kwn, -1, keepdims=True)) + 1e-8)
        kn_s[...] = kn.reshape(_B * _KW, _DT).astype(_bf16)
        m_s[...] = jnp.full((_B * _KW, 1), -jnp.inf, _f32)
        den_s[...] = jnp.zeros((_B * _KW, 1), _f32)
        acc_s[...] = jnp.zeros((_B * _KW, _DT), _f32)

    te_c = te_ref[...]
    teb = te_c.astype(_bf16)
    tinv = 1.0 / (jnp.sqrt(jnp.sum(te_c * te_c, -1, keepdims=True)) + 1e-8)
    cos = lax.dot_general(kn_s[...], teb, _CN,
                          preferred_element_type=_f32) * tinv.reshape(1, _VC)
    m_new = jnp.maximum(m_s[...], jnp.max(cos, -1, keepdims=True))
    scale = jnp.exp(m_s[...] - m_new)
    e = jnp.exp(cos - m_new)
    den_s[...] = den_s[...] * scale + jnp.sum(e, -1, keepdims=True)
    acc_s[...] = acc_s[...] * scale + lax.dot_general(
        e.astype(_bf16), teb, _CM, preferred_element_type=_f32)
    m_s[...] = m_new

    @pl.when(v == _NVC - 1)
    def _fin():
        out_ref[...] = (acc_s[...] / den_s[...]).reshape(_B, _KW, _DT)


def _const(shape):
    nd = len(shape)
    return pl.BlockSpec(shape, lambda b: (0,) * nd)


def _whole(shape):
    nd = len(shape)
    return pl.BlockSpec(shape, lambda: (0,) * nd)


@functools.partial(jax.jit)
def kernel(audio_feat, params, token_emb):
    p = params
    cls9 = jnp.concatenate([p['parallel_cls'][0], p['cascaded_cls'][0]], axis=0)
    cls16 = jnp.pad(cls9, ((0, _SQ - _NQ), (0, 0)))
    row = lambda a: a.reshape(1, -1)

    c_all = pl.pallas_call(
        _ctx_kernel,
        grid=(_B // _BB,),
        in_specs=[
            pl.BlockSpec((_BB, _T, _DA), lambda b: (b, 0, 0)),
            _const((_SQ, _DA)),
            _const((_DA, _DA)), _const((1, _DA)),
            _const((_DA, _DA)),
        ],
        out_specs=pl.BlockSpec((_BB, _R, _DA), lambda b: (b, 0, 0)),
        out_shape=jax.ShapeDtypeStruct((_B, _R, _DA), _bf16),
        scratch_shapes=[pltpu.VMEM((_R, _DA), _bf16),
                        pltpu.VMEM((_R, _SQ), _f32)],
        compiler_params=pltpu.CompilerParams(
            dimension_semantics=("arbitrary",)),
    )(audio_feat, cls16, p['Wq'], row(p['bq']), p['Wk'])

    p_out, kw_raw = pl.pallas_call(
        _tail_kernel,
        in_specs=[
            _whole((_B, _R, _DA)),
            _whole((_SQ, _DA)),
            _whole((_DA, _DA)), _whole((1, _DA)),
            _whole((_DA, _DA)), _whole((1, _DA)),
            _whole((1, _DA)), _whole((1, _DA)),
            _whole((_DA, _FF)), _whole((1, _FF)),
            _whole((_FF, _DA)), _whole((1, _DA)),
            _whole((1, _DA)), _whole((1, _DA)),
            _whole((_DA, _DT)), _whole((1, _DT)),
            _whole((_DA, _DT)), _whole((1, _DT)),
        ],
        out_specs=[_whole((_B, 1, _DT)), _whole((_B, _KW, _DT))],
        out_shape=[jax.ShapeDtypeStruct((_B, 1, _DT), _f32),
                   jax.ShapeDtypeStruct((_B, _KW, _DT), _f32)],
    )(c_all, cls16,
      p['Wv'], row(p['bv']), p['Wo'], row(p['bo']),
      row(p['ln1_g']), row(p['ln1_b']),
      p['ffn_W1'], row(p['ffn_b1']), p['ffn_W2'], row(p['ffn_b2']),
      row(p['ln2_g']), row(p['ln2_b']),
      p['proj_W'], row(p['proj_b']), p['pproj_W'], row(p['pproj_b']))

    keywords = pl.pallas_call(
        _vq_kernel,
        grid=(_NVC,),
        in_specs=[
            _const((_B, _KW, _DT)),
            _const((1, 1, _DT)), _const((1, 1, _DT)),
            pl.BlockSpec((_VC, _DT), lambda v: (v, 0)),
        ],
        out_specs=_const((_B, _KW, _DT)),
        out_shape=jax.ShapeDtypeStruct((_B, _KW, _DT), _f32),
        scratch_shapes=[pltpu.VMEM((_B * _KW, _DT), _bf16),
                        pltpu.VMEM((_B * _KW, 1), _f32),
                        pltpu.VMEM((_B * _KW, 1), _f32),
                        pltpu.VMEM((_B * _KW, _DT), _f32)],
        compiler_params=pltpu.CompilerParams(
            dimension_semantics=("arbitrary",)),
    )(kw_raw, p['bn_g'].reshape(1, 1, _DT), p['bn_b'].reshape(1, 1, _DT),
      token_emb)

    return jnp.concatenate([p_out, keywords], axis=1)


# speedup vs baseline: 9.2015x; 1.1293x over previous
import functools

import jax
import jax.numpy as jnp
from jax import lax
from jax.experimental import pallas as pl
from jax.experimental.pallas import tpu as pltpu

_B, _T, _DA = 16, 512, 768
_KW, _DT, _VOCAB = 8, 512, 8192
_H, _DH, _FF = 12, 64, 3072
_EPS = 1e-5
_SQ = 16
_NQ = 1 + _KW
_R = _H * _SQ
_BB = 2
_NC = _B // _BB

_bf16 = jnp.bfloat16
_f32 = jnp.float32

_CN = (((1,), (1,)), ((), ()))
_CM = (((1,), (0,)), ((), ()))


def _ln(x, g, b):
    m = jnp.mean(x, axis=-1, keepdims=True)
    v = jnp.mean((x - m) ** 2, axis=-1, keepdims=True)
    return (x - m) / jnp.sqrt(v + _EPS) * g + b


def _head_mask(shape, row_axis, col_axis):
    return (lax.broadcasted_iota(jnp.int32, shape, col_axis) // _DH
            == lax.broadcasted_iota(jnp.int32, shape, row_axis))


def _main_kernel(a_ref, cls_ref, wq_ref, bq_ref, wk_ref,
                 wv_hbm, wo_hbm, w1_hbm, w2_hbm,
                 bv_ref, bo_ref, g1_ref, be1_ref, b1_ref, b2_ref,
                 g2_ref, be2_ref, pjw_ref, pjb_ref, ppw_ref, ppb_ref,
                 p_out_ref, kwr_ref,
                 qzk_s, s1_s, c_s, wv_s, wo_s, w1_s, w2_s,
                 sem_v, sem_o, sem_1, sem_2):
    i = pl.program_id(0)

    @pl.when(i == 0)
    def _init():
        pltpu.make_async_copy(wv_hbm, wv_s, sem_v).start()
        pltpu.make_async_copy(wo_hbm, wo_s, sem_o).start()
        pltpu.make_async_copy(w1_hbm, w1_s, sem_1).start()
        pltpu.make_async_copy(w2_hbm, w2_s, sem_2).start()
        cls = cls_ref[...]
        q = (jnp.dot(cls, wq_ref[...], preferred_element_type=_f32)
             + bq_ref[...]) * (1.0 / 8.0)
        hm = _head_mask((_H, 1, _DA), 0, 2)
        qz = jnp.where(hm, jnp.broadcast_to(q[None], (_H, _SQ, _DA)), 0.0)
        qz = qz.reshape(_R, _DA).astype(_bf16)
        qzk = lax.dot_general(qz, wk_ref[...].astype(_bf16), _CN,
                              preferred_element_type=_f32)
        qzk_s[...] = qzk.astype(_bf16)
        s1_s[...] = lax.dot_general(qzk_s[...], cls.astype(_bf16), _CN,
                                    preferred_element_type=_f32)

    @pl.when(i < _NC)
    def _ctx():
        qzk = qzk_s[...]
        s1 = s1_s[...][:, : _NQ]
        m1 = jnp.max(s1, -1, keepdims=True)
        clsx = cls_ref[: _NQ].astype(_bf16)
        for j in range(_BB):
            xa = a_ref[j].astype(_bf16)
            s2 = lax.dot_general(qzk, xa, _CN, preferred_element_type=_f32)
            m = jnp.maximum(m1, jnp.max(s2, -1, keepdims=True))
            e1 = jnp.exp(s1 - m)
            e2 = jnp.exp(s2 - m)
            den = (jnp.sum(e1, -1, keepdims=True)
                   + jnp.sum(e2, -1, keepdims=True))
            c = (lax.dot_general(e1.astype(_bf16), clsx, _CM,
                                 preferred_element_type=_f32)
                 + lax.dot_general(e2.astype(_bf16), xa, _CM,
                                   preferred_element_type=_f32)) / den
            b = i * _BB + j
            c_s[pl.ds(b * _R, _R), :] = c.astype(_bf16)

    @pl.when(i == _NC)
    def _tail():
        pltpu.make_async_copy(wv_hbm, wv_s, sem_v).wait()
        pltpu.make_async_copy(wo_hbm, wo_s, sem_o).wait()
        pltpu.make_async_copy(w1_hbm, w1_s, sem_1).wait()
        pltpu.make_async_copy(w2_hbm, w2_s, sem_2).wait()
        wvb = wv_s[...].astype(_bf16)
        hm4 = _head_mask((1, _H, 1, _DA), 1, 3)
        halves = []
        hb = _B // 2
        for k in range(2):
            c2 = c_s[pl.ds(k * hb * _R, hb * _R), :]
            cw = lax.dot_general(c2, wvb, _CM,
                                 preferred_element_type=_f32)
            halves.append(jnp.sum(
                jnp.where(hm4, cw.reshape(hb, _H, _SQ, _DA), 0.0), axis=1))
        o = jnp.concatenate(halves, axis=0)
        o2 = o.reshape(_B * _SQ, _DA) + bv_ref[...]
        cls256 = jnp.broadcast_to(cls_ref[None], (_B, _SQ, _DA)).reshape(
            _B * _SQ, _DA)
        x1 = cls256 + jnp.dot(o2.astype(_bf16), wo_s[...].astype(_bf16),
                              preferred_element_type=_f32) + bo_ref[...]
        xn = _ln(x1, g1_ref[...], be1_ref[...])
        h = jax.nn.gelu(jnp.dot(xn.astype(_bf16), w1_s[...].astype(_bf16),
                                preferred_element_type=_f32) + b1_ref[...])
        x2 = xn + jnp.dot(h.astype(_bf16), w2_s[...].astype(_bf16),
                          preferred_element_type=_f32) + b2_ref[...]
        xo = _ln(x2, g2_ref[...], be2_ref[...])
        xob = xo.astype(_bf16)
        yp = jnp.dot(xob, ppw_ref[...].astype(_bf16),
                     preferred_element_type=_f32) + ppb_ref[...]
        ykw = jnp.dot(xob, pjw_ref[...].astype(_bf16),
                      preferred_element_type=_f32) + pjb_ref[...]
        p_out_ref[...] = yp.reshape(_B, _SQ, _DT)[:, 0:1, :]
        kwr_ref[...] = ykw.reshape(_B, _SQ, _DT)[:, 1:_NQ, :]


_VC = 1024
_NVC = _VOCAB // _VC


def _vq_kernel(kwr_ref, bng_ref, bnb_ref, te_ref, out_ref,
               kn_s, m_s, den_s, acc_s):
    v = pl.program_id(0)

    @pl.when(v == 0)
    def _init():
        kw = kwr_ref[...]
        mu = jnp.mean(kw, axis=0, keepdims=True)
        var = jnp.mean((kw - mu) ** 2, axis=0, keepdims=True)
        kwn = (kw - mu) / jnp.sqrt(var + _EPS) * bng_ref[...] + bnb_ref[...]
        kn = kwn / (jnp.sqrt(jnp.sum(kwn * kwn, -1, keepdims=True)) + 1e-8)
        kn_s[...] = kn.reshape(_B * _KW, _DT).astype(_bf16)
        m_s[...] = jnp.full((_B * _KW, 1), -jnp.inf, _f32)
        den_s[...] = jnp.zeros((_B * _KW, 1), _f32)
        acc_s[...] = jnp.zeros((_B * _KW, _DT), _f32)

    te_c = te_ref[...]
    teb = te_c.astype(_bf16)
    tinv = 1.0 / (jnp.sqrt(jnp.sum(te_c * te_c, -1, keepdims=True)) + 1e-8)
    cos = lax.dot_general(kn_s[...], teb, _CN,
                          preferred_element_type=_f32) * tinv.reshape(1, _VC)
    m_new = jnp.maximum(m_s[...], jnp.max(cos, -1, keepdims=True))
    scale = jnp.exp(m_s[...] - m_new)
    e = jnp.exp(cos - m_new)
    den_s[...] = den_s[...] * scale + jnp.sum(e, -1, keepdims=True)
    acc_s[...] = acc_s[...] * scale + lax.dot_general(
        e.astype(_bf16), teb, _CM, preferred_element_type=_f32)
    m_s[...] = m_new

    @pl.when(v == _NVC - 1)
    def _fin():
        out_ref[...] = (acc_s[...] / den_s[...]).reshape(_B, _KW, _DT)


def _const(shape):
    nd = len(shape)
    return pl.BlockSpec(shape, lambda b: (0,) * nd)


@functools.partial(jax.jit)
def kernel(audio_feat, params, token_emb):
    p = params
    cls9 = jnp.concatenate([p['parallel_cls'][0], p['cascaded_cls'][0]], axis=0)
    cls16 = jnp.pad(cls9, ((0, _SQ - _NQ), (0, 0)))
    row = lambda a: a.reshape(1, -1)
    hbm = pl.BlockSpec(memory_space=pltpu.MemorySpace.HBM)

    p_out, kw_raw = pl.pallas_call(
        _main_kernel,
        grid=(_NC + 1,),
        in_specs=[
            pl.BlockSpec((_BB, _T, _DA),
                         lambda i: (jnp.minimum(i, _NC - 1), 0, 0)),
            _const((_SQ, _DA)),
            _const((_DA, _DA)), _const((1, _DA)),
            _const((_DA, _DA)),
            hbm, hbm, hbm, hbm,
            _const((1, _DA)), _const((1, _DA)),
            _const((1, _DA)), _const((1, _DA)),
            _const((1, _FF)), _const((1, _DA)),
            _const((1, _DA)), _const((1, _DA)),
            _const((_DA, _DT)), _const((1, _DT)),
            _const((_DA, _DT)), _const((1, _DT)),
        ],
        out_specs=[_const((_B, 1, _DT)), _const((_B, _KW, _DT))],
        out_shape=[jax.ShapeDtypeStruct((_B, 1, _DT), _f32),
                   jax.ShapeDtypeStruct((_B, _KW, _DT), _f32)],
        scratch_shapes=[
            pltpu.VMEM((_R, _DA), _bf16),
            pltpu.VMEM((_R, _SQ), _f32),
            pltpu.VMEM((_B * _R, _DA), _bf16),
            pltpu.VMEM((_DA, _DA), _f32),
            pltpu.VMEM((_DA, _DA), _f32),
            pltpu.VMEM((_DA, _FF), _f32),
            pltpu.VMEM((_FF, _DA), _f32),
            pltpu.SemaphoreType.DMA,
            pltpu.SemaphoreType.DMA,
            pltpu.SemaphoreType.DMA,
            pltpu.SemaphoreType.DMA,
        ],
        compiler_params=pltpu.CompilerParams(
            dimension_semantics=("arbitrary",)),
    )(audio_feat, cls16, p['Wq'], row(p['bq']), p['Wk'],
      p['Wv'], p['Wo'], p['ffn_W1'], p['ffn_W2'],
      row(p['bv']), row(p['bo']),
      row(p['ln1_g']), row(p['ln1_b']),
      row(p['ffn_b1']), row(p['ffn_b2']),
      row(p['ln2_g']), row(p['ln2_b']),
      p['proj_W'], row(p['proj_b']), p['pproj_W'], row(p['pproj_b']))

    keywords = pl.pallas_call(
        _vq_kernel,
        grid=(_NVC,),
        in_specs=[
            _const((_B, _KW, _DT)),
            _const((1, 1, _DT)), _const((1, 1, _DT)),
            pl.BlockSpec((_VC, _DT), lambda v: (v, 0)),
        ],
        out_specs=_const((_B, _KW, _DT)),
        out_shape=jax.ShapeDtypeStruct((_B, _KW, _DT), _f32),
        scratch_shapes=[pltpu.VMEM((_B * _KW, _DT), _bf16),
                        pltpu.VMEM((_B * _KW, 1), _f32),
                        pltpu.VMEM((_B * _KW, 1), _f32),
                        pltpu.VMEM((_B * _KW, _DT), _f32)],
        compiler_params=pltpu.CompilerParams(
            dimension_semantics=("arbitrary",)),
    )(kw_raw, p['bn_g'].reshape(1, 1, _DT), p['bn_b'].reshape(1, 1, _DT),
      token_emb)

    return jnp.concatenate([p_out, keywords], axis=1)


# speedup vs baseline: 9.5821x; 1.0414x over previous
import functools

import jax
import jax.numpy as jnp
from jax import lax
from jax.experimental import pallas as pl
from jax.experimental.pallas import tpu as pltpu

_B, _T, _DA = 16, 512, 768
_KW, _DT, _VOCAB = 8, 512, 8192
_H, _DH, _FF = 12, 64, 3072
_EPS = 1e-5
_SQ = 16
_NQ = 1 + _KW
_R = _H * _SQ
_BB = 2
_NC = _B // _BB

_bf16 = jnp.bfloat16
_f32 = jnp.float32

_CN = (((1,), (1,)), ((), ()))
_CM = (((1,), (0,)), ((), ()))


def _ln(x, g, b):
    m = jnp.mean(x, axis=-1, keepdims=True)
    v = jnp.mean((x - m) ** 2, axis=-1, keepdims=True)
    return (x - m) / jnp.sqrt(v + _EPS) * g + b


def _head_mask(shape, row_axis, col_axis):
    return (lax.broadcasted_iota(jnp.int32, shape, col_axis) // _DH
            == lax.broadcasted_iota(jnp.int32, shape, row_axis))


_VC = 1024
_NVC = _VOCAB // _VC


def _main_kernel(a_ref, cls_ref, wq_ref, bq_ref, wk_ref,
                 wv_hbm, wo_hbm, w1_hbm, w2_hbm,
                 bv_ref, bo_ref, g1_ref, be1_ref, b1_ref, b2_ref,
                 g2_ref, be2_ref, pjw_ref, pjb_ref, ppw_ref, ppb_ref,
                 bng_ref, bnb_ref, te_ref,
                 p_out_ref, kws_out_ref,
                 qzk_s, s1_s, c_s, wv_s, wo_s, w1_s, w2_s,
                 kwr_s, kn_s, den_s, acc_s,
                 sem_v, sem_o, sem_1, sem_2):
    i = pl.program_id(0)

    @pl.when(i == 0)
    def _init():
        pltpu.make_async_copy(wv_hbm, wv_s, sem_v).start()
        pltpu.make_async_copy(wo_hbm, wo_s, sem_o).start()
        pltpu.make_async_copy(w1_hbm, w1_s, sem_1).start()
        pltpu.make_async_copy(w2_hbm, w2_s, sem_2).start()
        cls = cls_ref[...]
        q = (jnp.dot(cls, wq_ref[...], preferred_element_type=_f32)
             + bq_ref[...]) * (1.0 / 8.0)
        hm = _head_mask((_H, 1, _DA), 0, 2)
        qz = jnp.where(hm, jnp.broadcast_to(q[None], (_H, _SQ, _DA)), 0.0)
        qz = qz.reshape(_R, _DA).astype(_bf16)
        qzk = lax.dot_general(qz, wk_ref[...].astype(_bf16), _CN,
                              preferred_element_type=_f32)
        qzk_s[...] = qzk.astype(_bf16)
        s1_s[...] = lax.dot_general(qzk_s[...], cls.astype(_bf16), _CN,
                                    preferred_element_type=_f32)

    @pl.when(i < _NC)
    def _ctx():
        qzk = qzk_s[...]
        s1 = s1_s[...][:, : _NQ]
        m1 = jnp.max(s1, -1, keepdims=True)
        clsx = cls_ref[: _NQ].astype(_bf16)
        for j in range(_BB):
            xa = a_ref[j].astype(_bf16)
            s2 = lax.dot_general(qzk, xa, _CN, preferred_element_type=_f32)
            m = jnp.maximum(m1, jnp.max(s2, -1, keepdims=True))
            e1 = jnp.exp(s1 - m)
            e2 = jnp.exp(s2 - m)
            den = (jnp.sum(e1, -1, keepdims=True)
                   + jnp.sum(e2, -1, keepdims=True))
            c = (lax.dot_general(e1.astype(_bf16), clsx, _CM,
                                 preferred_element_type=_f32)
                 + lax.dot_general(e2.astype(_bf16), xa, _CM,
                                   preferred_element_type=_f32)) / den
            b = i * _BB + j
            c_s[pl.ds(b * _R, _R), :] = c.astype(_bf16)

    @pl.when(i == _NC)
    def _tail():
        pltpu.make_async_copy(wv_hbm, wv_s, sem_v).wait()
        pltpu.make_async_copy(wo_hbm, wo_s, sem_o).wait()
        pltpu.make_async_copy(w1_hbm, w1_s, sem_1).wait()
        pltpu.make_async_copy(w2_hbm, w2_s, sem_2).wait()
        wvb = wv_s[...].astype(_bf16)
        hm4 = _head_mask((1, _H, 1, _DA), 1, 3)
        halves = []
        hb = _B // 2
        for k in range(2):
            c2 = c_s[pl.ds(k * hb * _R, hb * _R), :]
            cw = lax.dot_general(c2, wvb, _CM,
                                 preferred_element_type=_f32)
            halves.append(jnp.sum(
                jnp.where(hm4, cw.reshape(hb, _H, _SQ, _DA), 0.0), axis=1))
        o = jnp.concatenate(halves, axis=0)
        o2 = o.reshape(_B * _SQ, _DA) + bv_ref[...]
        cls256 = jnp.broadcast_to(cls_ref[None], (_B, _SQ, _DA)).reshape(
            _B * _SQ, _DA)
        x1 = cls256 + jnp.dot(o2.astype(_bf16), wo_s[...].astype(_bf16),
                              preferred_element_type=_f32) + bo_ref[...]
        xn = _ln(x1, g1_ref[...], be1_ref[...])
        h = jax.nn.gelu(jnp.dot(xn.astype(_bf16), w1_s[...].astype(_bf16),
                                preferred_element_type=_f32) + b1_ref[...])
        x2 = xn + jnp.dot(h.astype(_bf16), w2_s[...].astype(_bf16),
                          preferred_element_type=_f32) + b2_ref[...]
        xo = _ln(x2, g2_ref[...], be2_ref[...])
        xob = xo.astype(_bf16)
        yp = jnp.dot(xob, ppw_ref[...].astype(_bf16),
                     preferred_element_type=_f32) + ppb_ref[...]
        ykw = jnp.dot(xob, pjw_ref[...].astype(_bf16),
                      preferred_element_type=_f32) + pjb_ref[...]
        p_out_ref[...] = yp.reshape(_B, _SQ, _DT)[:, 0:1, :]
        kwr_s[...] = ykw.reshape(_B, _SQ, _DT)[:, 1:_NQ, :]

    @pl.when(i == _NC + 1)
    def _bn():
        kw = kwr_s[...]
        mu = jnp.mean(kw, axis=0, keepdims=True)
        var = jnp.mean((kw - mu) ** 2, axis=0, keepdims=True)
        kwn = (kw - mu) / jnp.sqrt(var + _EPS) * bng_ref[...] + bnb_ref[...]
        kn = kwn / (jnp.sqrt(jnp.sum(kwn * kwn, -1, keepdims=True)) + 1e-8)
        kn_s[...] = kn.reshape(_B * _KW, _DT).astype(_bf16)
        den_s[...] = jnp.zeros((_B * _KW, 1), _f32)
        acc_s[...] = jnp.zeros((_B * _KW, _DT), _f32)

    @pl.when(i > _NC)
    def _vq():
        te_c = te_ref[...]
        teb = te_c.astype(_bf16)
        tinv = 1.0 / (jnp.sqrt(jnp.sum(te_c * te_c, -1, keepdims=True))
                      + 1e-8)
        cos = lax.dot_general(kn_s[...], teb, _CN,
                              preferred_element_type=_f32) * tinv.reshape(
                                  1, _VC)
        e = jnp.exp(cos)
        den_s[...] = den_s[...] + jnp.sum(e, -1, keepdims=True)
        acc_s[...] = acc_s[...] + lax.dot_general(
            e.astype(_bf16), teb, _CM, preferred_element_type=_f32)

    @pl.when(i == _NC + _NVC)
    def _fin():
        kws_out_ref[...] = (acc_s[...] / den_s[...]).reshape(_B, _KW, _DT)


def _const(shape):
    nd = len(shape)
    return pl.BlockSpec(shape, lambda b: (0,) * nd)


@functools.partial(jax.jit)
def kernel(audio_feat, params, token_emb):
    p = params
    cls9 = jnp.concatenate([p['parallel_cls'][0], p['cascaded_cls'][0]], axis=0)
    cls16 = jnp.pad(cls9, ((0, _SQ - _NQ), (0, 0)))
    row = lambda a: a.reshape(1, -1)
    hbm = pl.BlockSpec(memory_space=pltpu.MemorySpace.HBM)

    p_out, keywords = pl.pallas_call(
        _main_kernel,
        grid=(_NC + 1 + _NVC,),
        in_specs=[
            pl.BlockSpec((_BB, _T, _DA),
                         lambda i: (jnp.minimum(i, _NC - 1), 0, 0)),
            _const((_SQ, _DA)),
            _const((_DA, _DA)), _const((1, _DA)),
            _const((_DA, _DA)),
            hbm, hbm, hbm, hbm,
            _const((1, _DA)), _const((1, _DA)),
            _const((1, _DA)), _const((1, _DA)),
            _const((1, _FF)), _const((1, _DA)),
            _const((1, _DA)), _const((1, _DA)),
            _const((_DA, _DT)), _const((1, _DT)),
            _const((_DA, _DT)), _const((1, _DT)),
            _const((1, 1, _DT)), _const((1, 1, _DT)),
            pl.BlockSpec((_VC, _DT),
                         lambda i: (jnp.clip(i - _NC - 1, 0, _NVC - 1), 0)),
        ],
        out_specs=[_const((_B, 1, _DT)), _const((_B, _KW, _DT))],
        out_shape=[jax.ShapeDtypeStruct((_B, 1, _DT), _f32),
                   jax.ShapeDtypeStruct((_B, _KW, _DT), _f32)],
        scratch_shapes=[
            pltpu.VMEM((_R, _DA), _bf16),
            pltpu.VMEM((_R, _SQ), _f32),
            pltpu.VMEM((_B * _R, _DA), _bf16),
            pltpu.VMEM((_DA, _DA), _f32),
            pltpu.VMEM((_DA, _DA), _f32),
            pltpu.VMEM((_DA, _FF), _f32),
            pltpu.VMEM((_FF, _DA), _f32),
            pltpu.VMEM((_B, _KW, _DT), _f32),
            pltpu.VMEM((_B * _KW, _DT), _bf16),
            pltpu.VMEM((_B * _KW, 1), _f32),
            pltpu.VMEM((_B * _KW, _DT), _f32),
            pltpu.SemaphoreType.DMA,
            pltpu.SemaphoreType.DMA,
            pltpu.SemaphoreType.DMA,
            pltpu.SemaphoreType.DMA,
        ],
        compiler_params=pltpu.CompilerParams(
            dimension_semantics=("arbitrary",)),
    )(audio_feat, cls16, p['Wq'], row(p['bq']), p['Wk'],
      p['Wv'], p['Wo'], p['ffn_W1'], p['ffn_W2'],
      row(p['bv']), row(p['bo']),
      row(p['ln1_g']), row(p['ln1_b']),
      row(p['ffn_b1']), row(p['ffn_b2']),
      row(p['ln2_g']), row(p['ln2_b']),
      p['proj_W'], row(p['proj_b']), p['pproj_W'], row(p['pproj_b']),
      p['bn_g'].reshape(1, 1, _DT), p['bn_b'].reshape(1, 1, _DT),
      token_emb)

    return jnp.concatenate([p_out, keywords], axis=1)


# speedup vs baseline: 9.8405x; 1.0270x over previous
import functools

import jax
import jax.numpy as jnp
from jax import lax
from jax.experimental import pallas as pl
from jax.experimental.pallas import tpu as pltpu

_B, _T, _DA = 16, 512, 768
_KW, _DT, _VOCAB = 8, 512, 8192
_H, _DH, _FF = 12, 64, 3072
_EPS = 1e-5
_SQ = 16
_NQ = 1 + _KW
_R = _H * _SQ
_BB = 2
_NC = _B // _BB

_bf16 = jnp.bfloat16
_f32 = jnp.float32

_CN = (((1,), (1,)), ((), ()))
_CM = (((1,), (0,)), ((), ()))


def _ln(x, g, b):
    m = jnp.mean(x, axis=-1, keepdims=True)
    v = jnp.mean((x - m) ** 2, axis=-1, keepdims=True)
    return (x - m) / jnp.sqrt(v + _EPS) * g + b


def _head_mask(shape, row_axis, col_axis):
    return (lax.broadcasted_iota(jnp.int32, shape, col_axis) // _DH
            == lax.broadcasted_iota(jnp.int32, shape, row_axis))


_VC = 2048
_NVC = _VOCAB // _VC


def _main_kernel(a_ref, cls_ref, wq_ref, bq_ref, wk_ref,
                 wv_hbm, wo_hbm, w1_hbm, w2_hbm,
                 bv_ref, bo_ref, g1_ref, be1_ref, b1_ref, b2_ref,
                 g2_ref, be2_ref, pjw_ref, pjb_ref, ppw_ref, ppb_ref,
                 bng_ref, bnb_ref, te_ref,
                 p_out_ref, kws_out_ref,
                 qzk_s, s1_s, c_s, wv_s, wo_s, w1_s, w2_s,
                 kwr_s, kn_s, den_s, acc_s,
                 sem_v, sem_o, sem_1, sem_2):
    i = pl.program_id(0)

    @pl.when(i == 0)
    def _init():
        pltpu.make_async_copy(wv_hbm, wv_s, sem_v).start()
        pltpu.make_async_copy(wo_hbm, wo_s, sem_o).start()
        pltpu.make_async_copy(w1_hbm, w1_s, sem_1).start()
        pltpu.make_async_copy(w2_hbm, w2_s, sem_2).start()
        cls = cls_ref[...]
        q = (jnp.dot(cls, wq_ref[...], preferred_element_type=_f32)
             + bq_ref[...]) * (1.0 / 8.0)
        hm = _head_mask((_H, 1, _DA), 0, 2)
        qz = jnp.where(hm, jnp.broadcast_to(q[None], (_H, _SQ, _DA)), 0.0)
        qz = qz.reshape(_R, _DA).astype(_bf16)
        qzk = lax.dot_general(qz, wk_ref[...].astype(_bf16), _CN,
                              preferred_element_type=_f32)
        qzk_s[...] = qzk.astype(_bf16)
        s1_s[...] = lax.dot_general(qzk_s[...], cls.astype(_bf16), _CN,
                                    preferred_element_type=_f32)

    @pl.when(i < _NC)
    def _ctx():
        qzk = qzk_s[...]
        s1 = s1_s[...][:, : _NQ]
        m1 = jnp.max(s1, -1, keepdims=True)
        clsx = cls_ref[: _NQ].astype(_bf16)
        for j in range(_BB):
            xa = a_ref[j].astype(_bf16)
            s2 = lax.dot_general(qzk, xa, _CN, preferred_element_type=_f32)
            m = jnp.maximum(m1, jnp.max(s2, -1, keepdims=True))
            e1 = jnp.exp(s1 - m)
            e2 = jnp.exp(s2 - m)
            den = (jnp.sum(e1, -1, keepdims=True)
                   + jnp.sum(e2, -1, keepdims=True))
            c = (lax.dot_general(e1.astype(_bf16), clsx, _CM,
                                 preferred_element_type=_f32)
                 + lax.dot_general(e2.astype(_bf16), xa, _CM,
                                   preferred_element_type=_f32)) / den
            b = i * _BB + j
            c_s[pl.ds(b * _R, _R), :] = c.astype(_bf16)

    @pl.when(i == _NC)
    def _tail():
        pltpu.make_async_copy(wv_hbm, wv_s, sem_v).wait()
        pltpu.make_async_copy(wo_hbm, wo_s, sem_o).wait()
        pltpu.make_async_copy(w1_hbm, w1_s, sem_1).wait()
        pltpu.make_async_copy(w2_hbm, w2_s, sem_2).wait()
        wvb = wv_s[...].astype(_bf16)
        hm4 = _head_mask((1, _H, 1, _DA), 1, 3)
        halves = []
        hb = _B // 2
        for k in range(2):
            c2 = c_s[pl.ds(k * hb * _R, hb * _R), :]
            cw = lax.dot_general(c2, wvb, _CM,
                                 preferred_element_type=_f32)
            halves.append(jnp.sum(
                jnp.where(hm4, cw.reshape(hb, _H, _SQ, _DA), 0.0), axis=1))
        o = jnp.concatenate(halves, axis=0)
        o2 = o.reshape(_B * _SQ, _DA) + bv_ref[...]
        cls256 = jnp.broadcast_to(cls_ref[None], (_B, _SQ, _DA)).reshape(
            _B * _SQ, _DA)
        x1 = cls256 + jnp.dot(o2.astype(_bf16), wo_s[...].astype(_bf16),
                              preferred_element_type=_f32) + bo_ref[...]
        xn = _ln(x1, g1_ref[...], be1_ref[...])
        h = jax.nn.gelu(jnp.dot(xn.astype(_bf16), w1_s[...].astype(_bf16),
                                preferred_element_type=_f32) + b1_ref[...])
        x2 = xn + jnp.dot(h.astype(_bf16), w2_s[...].astype(_bf16),
                          preferred_element_type=_f32) + b2_ref[...]
        xo = _ln(x2, g2_ref[...], be2_ref[...])
        xob = xo.astype(_bf16)
        yp = jnp.dot(xob, ppw_ref[...].astype(_bf16),
                     preferred_element_type=_f32) + ppb_ref[...]
        ykw = jnp.dot(xob, pjw_ref[...].astype(_bf16),
                      preferred_element_type=_f32) + pjb_ref[...]
        p_out_ref[...] = yp.reshape(_B, _SQ, _DT)[:, 0:1, :]
        kwr_s[...] = ykw.reshape(_B, _SQ, _DT)[:, 1:_NQ, :]

    @pl.when(i == _NC + 1)
    def _bn():
        kw = kwr_s[...]
        mu = jnp.mean(kw, axis=0, keepdims=True)
        var = jnp.mean((kw - mu) ** 2, axis=0, keepdims=True)
        kwn = (kw - mu) / jnp.sqrt(var + _EPS) * bng_ref[...] + bnb_ref[...]
        kn = kwn / (jnp.sqrt(jnp.sum(kwn * kwn, -1, keepdims=True)) + 1e-8)
        kn_s[...] = kn.reshape(_B * _KW, _DT).astype(_bf16)
        den_s[...] = jnp.zeros((_B * _KW, 1), _f32)
        acc_s[...] = jnp.zeros((_B * _KW, _DT), _f32)

    @pl.when(i > _NC)
    def _vq():
        te_c = te_ref[...]
        teb = te_c.astype(_bf16)
        tinv = 1.0 / (jnp.sqrt(jnp.sum(te_c * te_c, -1, keepdims=True))
                      + 1e-8)
        cos = lax.dot_general(kn_s[...], teb, _CN,
                              preferred_element_type=_f32) * tinv.reshape(
                                  1, _VC)
        e = jnp.exp(cos)
        den_s[...] = den_s[...] + jnp.sum(e, -1, keepdims=True)
        acc_s[...] = acc_s[...] + lax.dot_general(
            e.astype(_bf16), teb, _CM, preferred_element_type=_f32)

    @pl.when(i == _NC + _NVC)
    def _fin():
        kws_out_ref[...] = (acc_s[...] / den_s[...]).reshape(_B, _KW, _DT)


def _const(shape):
    nd = len(shape)
    return pl.BlockSpec(shape, lambda b: (0,) * nd)


@functools.partial(jax.jit)
def kernel(audio_feat, params, token_emb):
    p = params
    cls9 = jnp.concatenate([p['parallel_cls'][0], p['cascaded_cls'][0]], axis=0)
    cls16 = jnp.pad(cls9, ((0, _SQ - _NQ), (0, 0)))
    row = lambda a: a.reshape(1, -1)
    hbm = pl.BlockSpec(memory_space=pltpu.MemorySpace.HBM)

    p_out, keywords = pl.pallas_call(
        _main_kernel,
        grid=(_NC + 1 + _NVC,),
        in_specs=[
            pl.BlockSpec((_BB, _T, _DA),
                         lambda i: (jnp.minimum(i, _NC - 1), 0, 0)),
            _const((_SQ, _DA)),
            _const((_DA, _DA)), _const((1, _DA)),
            _const((_DA, _DA)),
            hbm, hbm, hbm, hbm,
            _const((1, _DA)), _const((1, _DA)),
            _const((1, _DA)), _const((1, _DA)),
            _const((1, _FF)), _const((1, _DA)),
            _const((1, _DA)), _const((1, _DA)),
            _const((_DA, _DT)), _const((1, _DT)),
            _const((_DA, _DT)), _const((1, _DT)),
            _const((1, 1, _DT)), _const((1, 1, _DT)),
            pl.BlockSpec((_VC, _DT),
                         lambda i: (jnp.clip(i - _NC - 1, 0, _NVC - 1), 0)),
        ],
        out_specs=[_const((_B, 1, _DT)), _const((_B, _KW, _DT))],
        out_shape=[jax.ShapeDtypeStruct((_B, 1, _DT), _f32),
                   jax.ShapeDtypeStruct((_B, _KW, _DT), _f32)],
        scratch_shapes=[
            pltpu.VMEM((_R, _DA), _bf16),
            pltpu.VMEM((_R, _SQ), _f32),
            pltpu.VMEM((_B * _R, _DA), _bf16),
            pltpu.VMEM((_DA, _DA), _f32),
            pltpu.VMEM((_DA, _DA), _f32),
            pltpu.VMEM((_DA, _FF), _f32),
            pltpu.VMEM((_FF, _DA), _f32),
            pltpu.VMEM((_B, _KW, _DT), _f32),
            pltpu.VMEM((_B * _KW, _DT), _bf16),
            pltpu.VMEM((_B * _KW, 1), _f32),
            pltpu.VMEM((_B * _KW, _DT), _f32),
            pltpu.SemaphoreType.DMA,
            pltpu.SemaphoreType.DMA,
            pltpu.SemaphoreType.DMA,
            pltpu.SemaphoreType.DMA,
        ],
        compiler_params=pltpu.CompilerParams(
            dimension_semantics=("arbitrary",)),
    )(audio_feat, cls16, p['Wq'], row(p['bq']), p['Wk'],
      p['Wv'], p['Wo'], p['ffn_W1'], p['ffn_W2'],
      row(p['bv']), row(p['bo']),
      row(p['ln1_g']), row(p['ln1_b']),
      row(p['ffn_b1']), row(p['ffn_b2']),
      row(p['ln2_g']), row(p['ln2_b']),
      p['proj_W'], row(p['proj_b']), p['pproj_W'], row(p['pproj_b']),
      p['bn_g'].reshape(1, 1, _DT), p['bn_b'].reshape(1, 1, _DT),
      token_emb)

    return jnp.concatenate([p_out, keywords], axis=1)
